# Initial kernel scaffold; baseline (speedup 1.0000x reference)
import functools

import jax
import jax.numpy as jnp
from jax import lax
from jax.experimental import pallas as pl
from jax.experimental.pallas import tpu as pltpu
from jax.experimental.pallas import tpu_sc as plsc

N = 10000
E = 160000
IN = 256
H = 128
G = 64

NC = 2
NS = 16
NW = NC * NS
EPT = E // NW
CH = 128
NCH = (EPT + CH - 1) // CH
EPT_PAD = NCH * CH
STRIPE = N // NS
LANES = 16


def _sc_agg_body(h_hbm, src_hbm, dst_hbm, ew_hbm, out_hbm,
                 src_v, dst_v, ew_v, rows_v, agg_sh, sem):
    c = lax.axis_index("c")
    s = lax.axis_index("s")
    w = c * NS + s

    def zero_row(i, carry):
        for f in range(H // LANES):
            rows_v[i, pl.ds(f * LANES, LANES)] = jnp.zeros((LANES,), jnp.float32)
        return carry
    lax.fori_loop(0, CH, zero_row, 0)
    for k in range(5):
        pltpu.sync_copy(rows_v.at[pl.ds(0, 125), :],
                        agg_sh.at[pl.ds(s * STRIPE + k * 125, 125), :])

    pltpu.sync_copy(src_hbm.at[w], src_v)
    pltpu.sync_copy(dst_hbm.at[w], dst_v)
    pltpu.sync_copy(ew_hbm.at[w], ew_v)

    plsc.subcore_barrier()

    def chunk(j, carry):
        pltpu.async_copy(h_hbm.at[src_v.at[j]], rows_v, sem).wait()

        def edge(e, carry2):
            wgt = jnp.full((LANES,), ew_v[j, e], jnp.float32)
            for f in range(H // LANES):
                sl = pl.ds(f * LANES, LANES)
                rows_v[e, sl] = rows_v[e, sl] * wgt
            return carry2
        lax.fori_loop(0, CH, edge, 0, unroll=2)

        pltpu.sync_copy(rows_v, agg_sh.at[dst_v.at[j]], add=True)
        return carry
    lax.fori_loop(0, NCH, chunk, 0)

    plsc.subcore_barrier()

    pltpu.sync_copy(agg_sh.at[pl.ds(s * STRIPE, STRIPE), :],
                    out_hbm.at[c, pl.ds(s * STRIPE, STRIPE), :])


_sc_agg = pl.kernel(
    _sc_agg_body,
    out_type=jax.ShapeDtypeStruct((NC, N, H), jnp.float32),
    mesh=plsc.VectorSubcoreMesh(core_axis_name="c", subcore_axis_name="s"),
    scratch_types=[
        pltpu.VMEM((NCH, CH), jnp.int32),
        pltpu.VMEM((NCH, CH), jnp.int32),
        pltpu.VMEM((NCH, CH), jnp.float32),
        pltpu.VMEM((CH, H), jnp.float32),
        pltpu.VMEM_SHARED((N, H), jnp.float32),
        pltpu.SemaphoreType.DMA,
    ],
)


def _dot_t(a, b):
    return lax.dot_general(a, b, (((1,), (1,)), ((), ())),
                           preferred_element_type=jnp.float32)


def _embed_body(x_ref, w_ref, b_ref, o_ref):
    o_ref[...] = _dot_t(x_ref[...], w_ref[...]) + b_ref[...]


_embed = pl.pallas_call(
    _embed_body,
    out_shape=jax.ShapeDtypeStruct((N, H), jnp.float32),
)


def _conv_body(parts_ref, h_ref, wr_ref, br_ref, wroot_ref, g_ref, b_ref,
               o_ref):
    agg = parts_ref[0] + parts_ref[1]
    z = _dot_t(agg, wr_ref[...]) + _dot_t(h_ref[...], wroot_ref[...])
    z = z + br_ref[...]
    m = jnp.mean(z, axis=0, keepdims=True)
    v = jnp.mean((z - m) ** 2, axis=0, keepdims=True)
    o_ref[...] = jnp.maximum(
        (z - m) * lax.rsqrt(v + 1e-5) * g_ref[...] + b_ref[...], 0.0)


_conv = pl.pallas_call(
    _conv_body,
    out_shape=jax.ShapeDtypeStruct((N, H), jnp.float32),
)


def _final_body(parts_ref, h_ref, wr_ref, br_ref, wroot_ref, g_ref, b_ref,
                batch_ref, p1w_ref, p1b_ref, p2w_ref, p2b_ref, p3w_ref,
                p3b_ref, o_ref):
    agg = parts_ref[0] + parts_ref[1]
    z = _dot_t(agg, wr_ref[...]) + _dot_t(h_ref[...], wroot_ref[...])
    z = z + br_ref[...]
    m = jnp.mean(z, axis=0, keepdims=True)
    v = jnp.mean((z - m) ** 2, axis=0, keepdims=True)
    h2 = jnp.maximum(
        (z - m) * lax.rsqrt(v + 1e-5) * g_ref[...] + b_ref[...], 0.0)

    gids = lax.broadcasted_iota(jnp.int32, (N, G), 1)
    oh = (batch_ref[...] == gids).astype(jnp.float32)
    sums = lax.dot_general(oh, h2, (((0,), (0,)), ((), ())),
                           preferred_element_type=jnp.float32)
    cnt = jnp.sum(oh, axis=0)[:, None]
    gx = sums / jnp.maximum(cnt, 1.0)

    o = jnp.maximum(_dot_t(gx, p1w_ref[...]) + p1b_ref[...], 0.0)
    o = jnp.maximum(_dot_t(o, p2w_ref[...]) + p2b_ref[...], 0.0)
    o_ref[...] = _dot_t(o, p3w_ref[...]) + p3b_ref[...]


_final = pl.pallas_call(
    _final_body,
    out_shape=jax.ShapeDtypeStruct((G, 1), jnp.float32),
)


def kernel(x, edge_index, edge_attr, batch, emb_W, emb_b, c1_Wr, c1_br,
           c1_Wroot, bn1_g, bn1_b, c2_Wr, c2_br, c2_Wroot, bn2_g, bn2_b,
           p1_W, p1_b, p2_W, p2_b, p3_W, p3_b):
    pad = ((0, 0), (0, EPT_PAD - EPT))
    src3 = jnp.pad(edge_index[0].reshape(NW, EPT), pad).reshape(NW, NCH, CH)
    dst3 = jnp.pad(edge_index[1].reshape(NW, EPT), pad).reshape(NW, NCH, CH)
    ew3 = jnp.pad(edge_attr.reshape(NW, EPT), pad).reshape(NW, NCH, CH)

    batch2 = batch.reshape(N, 1)
    emb_b2 = emb_b.reshape(1, H)
    c1_br2 = c1_br.reshape(1, H)
    c2_br2 = c2_br.reshape(1, H)
    bn1_g2 = bn1_g.reshape(1, H)
    bn1_b2 = bn1_b.reshape(1, H)
    bn2_g2 = bn2_g.reshape(1, H)
    bn2_b2 = bn2_b.reshape(1, H)
    p1_b2 = p1_b.reshape(1, 2 * H)
    p2_b2 = p2_b.reshape(1, H)
    p3_b2 = p3_b.reshape(1, 1)

    h0 = _embed(x, emb_W, emb_b2)
    parts1 = _sc_agg(h0, src3, dst3, ew3)
    h1 = _conv(parts1, h0, c1_Wr, c1_br2, c1_Wroot, bn1_g2, bn1_b2)
    parts2 = _sc_agg(h1, src3, dst3, ew3)
    out = _final(parts2, h1, c2_Wr, c2_br2, c2_Wroot, bn2_g2, bn2_b2,
                 batch2, p1_W, p1_b2, p2_W, p2_b2, p3_W, p3_b2)
    return out


# speedup vs baseline: 3.2463x; 3.2463x over previous
import functools

import jax
import jax.numpy as jnp
from jax import lax
from jax.experimental import pallas as pl
from jax.experimental.pallas import tpu as pltpu
from jax.experimental.pallas import tpu_sc as plsc

N = 10000
E = 160000
IN = 256
H = 128
G = 64

NC = 2
NS = 16
NW = NC * NS
EPT = E // NW
CH = 128
NCH = (EPT + CH - 1) // CH
EPT_PAD = NCH * CH
NP = 10240
STRIPE = NP // NS
LANES = 16


def _sc_agg_body(h_hbm, src_hbm, dst_hbm, ew_hbm, out_hbm,
                 src_v, dst_v, ew_v, rows_v, agg_sh, sem):
    c = lax.axis_index("c")
    s = lax.axis_index("s")
    w = c * NS + s

    def zero_row(i, carry):
        for f in range(H // LANES):
            rows_v[i, pl.ds(f * LANES, LANES)] = jnp.zeros((LANES,), jnp.float32)
        return carry
    lax.fori_loop(0, CH, zero_row, 0)
    for k in range(STRIPE // CH):
        pltpu.sync_copy(rows_v,
                        agg_sh.at[pl.ds(s * STRIPE + k * CH, CH), :])

    pltpu.sync_copy(src_hbm.at[w], src_v)
    pltpu.sync_copy(dst_hbm.at[w], dst_v)
    pltpu.sync_copy(ew_hbm.at[w], ew_v)

    plsc.subcore_barrier()

    def chunk(j, carry):
        pltpu.async_copy(h_hbm.at[src_v.at[j]], rows_v, sem).wait()

        def egroup(g, carry2):
            ew16 = ew_v[j, pl.ds(g * LANES, LANES)]
            for l in range(LANES):
                wgt = jnp.full((LANES,), ew16[l], jnp.float32)
                e = g * LANES + l
                for f in range(H // LANES):
                    sl = pl.ds(f * LANES, LANES)
                    rows_v[e, sl] = rows_v[e, sl] * wgt
            return carry2
        lax.fori_loop(0, CH // LANES, egroup, 0)

        pltpu.sync_copy(rows_v, agg_sh.at[dst_v.at[j]], add=True)
        return carry
    lax.fori_loop(0, NCH, chunk, 0)

    plsc.subcore_barrier()

    pltpu.sync_copy(agg_sh.at[pl.ds(s * STRIPE, STRIPE), :],
                    out_hbm.at[c, pl.ds(s * STRIPE, STRIPE), :])


_sc_agg = pl.kernel(
    _sc_agg_body,
    out_type=jax.ShapeDtypeStruct((NC, NP, H), jnp.float32),
    mesh=plsc.VectorSubcoreMesh(core_axis_name="c", subcore_axis_name="s"),
    scratch_types=[
        pltpu.VMEM((NCH, CH), jnp.int32),
        pltpu.VMEM((NCH, CH), jnp.int32),
        pltpu.VMEM((NCH, CH), jnp.float32),
        pltpu.VMEM((CH, H), jnp.float32),
        pltpu.VMEM_SHARED((NP, H), jnp.float32),
        pltpu.SemaphoreType.DMA,
    ],
)


def _dot_t(a, b):
    return lax.dot_general(a, b, (((1,), (1,)), ((), ())),
                           preferred_element_type=jnp.float32)


def _embed_body(x_ref, w_ref, b_ref, o_ref):
    o_ref[...] = _dot_t(x_ref[...], w_ref[...]) + b_ref[...]


_embed = pl.pallas_call(
    _embed_body,
    out_shape=jax.ShapeDtypeStruct((N, H), jnp.float32),
)


def _conv_body(parts_ref, h_ref, wr_ref, br_ref, wroot_ref, g_ref, b_ref,
               o_ref):
    agg = parts_ref[0, :N, :] + parts_ref[1, :N, :]
    z = _dot_t(agg, wr_ref[...]) + _dot_t(h_ref[...], wroot_ref[...])
    z = z + br_ref[...]
    m = jnp.mean(z, axis=0, keepdims=True)
    v = jnp.mean((z - m) ** 2, axis=0, keepdims=True)
    o_ref[...] = jnp.maximum(
        (z - m) * lax.rsqrt(v + 1e-5) * g_ref[...] + b_ref[...], 0.0)


_conv = pl.pallas_call(
    _conv_body,
    out_shape=jax.ShapeDtypeStruct((N, H), jnp.float32),
)


def _final_body(parts_ref, h_ref, wr_ref, br_ref, wroot_ref, g_ref, b_ref,
                batch_ref, p1w_ref, p1b_ref, p2w_ref, p2b_ref, p3w_ref,
                p3b_ref, o_ref):
    agg = parts_ref[0, :N, :] + parts_ref[1, :N, :]
    z = _dot_t(agg, wr_ref[...]) + _dot_t(h_ref[...], wroot_ref[...])
    z = z + br_ref[...]
    m = jnp.mean(z, axis=0, keepdims=True)
    v = jnp.mean((z - m) ** 2, axis=0, keepdims=True)
    h2 = jnp.maximum(
        (z - m) * lax.rsqrt(v + 1e-5) * g_ref[...] + b_ref[...], 0.0)

    gids = lax.broadcasted_iota(jnp.int32, (N, G), 1)
    oh = (batch_ref[...] == gids).astype(jnp.float32)
    sums = lax.dot_general(oh, h2, (((0,), (0,)), ((), ())),
                           preferred_element_type=jnp.float32)
    cnt = jnp.sum(oh, axis=0)[:, None]
    gx = sums / jnp.maximum(cnt, 1.0)

    o = jnp.maximum(_dot_t(gx, p1w_ref[...]) + p1b_ref[...], 0.0)
    o = jnp.maximum(_dot_t(o, p2w_ref[...]) + p2b_ref[...], 0.0)
    o_ref[...] = (jnp.sum(o * p3w_ref[...], axis=1, keepdims=True)
                  + p3b_ref[...])


_final = pl.pallas_call(
    _final_body,
    out_shape=jax.ShapeDtypeStruct((G, 1), jnp.float32),
)


def kernel(x, edge_index, edge_attr, batch, emb_W, emb_b, c1_Wr, c1_br,
           c1_Wroot, bn1_g, bn1_b, c2_Wr, c2_br, c2_Wroot, bn2_g, bn2_b,
           p1_W, p1_b, p2_W, p2_b, p3_W, p3_b):
    pad = ((0, 0), (0, EPT_PAD - EPT))
    src3 = jnp.pad(edge_index[0].reshape(NW, EPT), pad).reshape(NW, NCH, CH)
    dst3 = jnp.pad(edge_index[1].reshape(NW, EPT), pad).reshape(NW, NCH, CH)
    ew3 = jnp.pad(edge_attr.reshape(NW, EPT), pad).reshape(NW, NCH, CH)

    batch2 = batch.reshape(N, 1)
    emb_b2 = emb_b.reshape(1, H)
    c1_br2 = c1_br.reshape(1, H)
    c2_br2 = c2_br.reshape(1, H)
    bn1_g2 = bn1_g.reshape(1, H)
    bn1_b2 = bn1_b.reshape(1, H)
    bn2_g2 = bn2_g.reshape(1, H)
    bn2_b2 = bn2_b.reshape(1, H)
    p1_b2 = p1_b.reshape(1, 2 * H)
    p2_b2 = p2_b.reshape(1, H)
    p3_b2 = p3_b.reshape(1, 1)

    h0 = _embed(x, emb_W, emb_b2)
    parts1 = _sc_agg(h0, src3, dst3, ew3)
    h1 = _conv(parts1, h0, c1_Wr, c1_br2, c1_Wroot, bn1_g2, bn1_b2)
    parts2 = _sc_agg(h1, src3, dst3, ew3)
    out = _final(parts2, h1, c2_Wr, c2_br2, c2_Wroot, bn2_g2, bn2_b2,
                 batch2, p1_W, p1_b2, p2_W, p2_b2, p3_W, p3_b2)
    return out


# speedup vs baseline: 3.5702x; 1.0998x over previous
import functools

import jax
import jax.numpy as jnp
from jax import lax
from jax.experimental import pallas as pl
from jax.experimental.pallas import tpu as pltpu
from jax.experimental.pallas import tpu_sc as plsc

N = 10000
E = 160000
IN = 256
H = 128
G = 64

NC = 2
NS = 16
NW = NC * NS
EPT = E // NW
CH = 64
NCH = (EPT + CH - 1) // CH
NCH += NCH % 2
EPT_PAD = NCH * CH
NP = 10240
STRIPE = NP // NS
LANES = 16


def _sc_agg_body(h_hbm, src_hbm, dst_hbm, ew_hbm, out_hbm,
                 src_v, dst_v, ew_v, gb0, gb1, agg_sh,
                 sem_g0, sem_g1, sem_s0, sem_s1):
    c = lax.axis_index("c")
    s = lax.axis_index("s")
    w = c * NS + s

    def zero_row(i, carry):
        for f in range(H // LANES):
            gb0[i, pl.ds(f * LANES, LANES)] = jnp.zeros((LANES,), jnp.float32)
        return carry
    lax.fori_loop(0, CH, zero_row, 0)
    for k in range(STRIPE // CH):
        pltpu.sync_copy(gb0,
                        agg_sh.at[pl.ds(s * STRIPE + k * CH, CH), :])

    pltpu.sync_copy(src_hbm.at[w], src_v)
    pltpu.sync_copy(dst_hbm.at[w], dst_v)
    pltpu.sync_copy(ew_hbm.at[w], ew_v)

    plsc.subcore_barrier()

    def scale(j, buf):
        def egroup(g, carry2):
            ew16 = ew_v[j, pl.ds(g * LANES, LANES)]
            for l in range(LANES):
                wgt = jnp.full((LANES,), ew16[l], jnp.float32)
                e = g * LANES + l
                for f in range(H // LANES):
                    sl = pl.ds(f * LANES, LANES)
                    buf[e, sl] = buf[e, sl] * wgt
            return carry2
        lax.fori_loop(0, CH // LANES, egroup, 0)

    def start_gather(j, gbuf, sem):
        pltpu.async_copy(h_hbm.at[src_v.at[j]], gbuf, sem)

    def wait_gather(j, gbuf, sem):
        pltpu.make_async_copy(h_hbm.at[src_v.at[j]], gbuf, sem).wait()

    def start_scatter(j, sbuf, sem):
        pltpu.async_copy(sbuf, agg_sh.at[dst_v.at[j]], sem, add=True)

    def wait_scatter(j, sbuf, sem):
        pltpu.make_async_copy(sbuf, agg_sh.at[dst_v.at[j]], sem).wait()

    start_gather(0, gb0, sem_g0)
    start_gather(1, gb1, sem_g1)

    def pair(t, carry):
        j0 = 2 * t
        j1 = j0 + 1

        wait_gather(j0, gb0, sem_g0)
        scale(j0, gb0)
        start_scatter(j0, gb0, sem_s0)

        wait_gather(j1, gb1, sem_g1)
        scale(j1, gb1)
        start_scatter(j1, gb1, sem_s1)

        @pl.when(t < NCH // 2 - 1)
        def _():
            wait_scatter(j0, gb0, sem_s0)
            start_gather(j0 + 2, gb0, sem_g0)
            wait_scatter(j1, gb1, sem_s1)
            start_gather(j1 + 2, gb1, sem_g1)
        return carry
    lax.fori_loop(0, NCH // 2, pair, 0)

    wait_scatter(NCH - 2, gb0, sem_s0)
    wait_scatter(NCH - 1, gb1, sem_s1)

    plsc.subcore_barrier()

    pltpu.sync_copy(agg_sh.at[pl.ds(s * STRIPE, STRIPE), :],
                    out_hbm.at[c, pl.ds(s * STRIPE, STRIPE), :])


_sc_agg = pl.kernel(
    _sc_agg_body,
    out_type=jax.ShapeDtypeStruct((NC, NP, H), jnp.float32),
    mesh=plsc.VectorSubcoreMesh(core_axis_name="c", subcore_axis_name="s"),
    scratch_types=[
        pltpu.VMEM((NCH, CH), jnp.int32),
        pltpu.VMEM((NCH, CH), jnp.int32),
        pltpu.VMEM((NCH, CH), jnp.float32),
        pltpu.VMEM((CH, H), jnp.float32),
        pltpu.VMEM((CH, H), jnp.float32),
        pltpu.VMEM_SHARED((NP, H), jnp.float32),
        pltpu.SemaphoreType.DMA,
        pltpu.SemaphoreType.DMA,
        pltpu.SemaphoreType.DMA,
        pltpu.SemaphoreType.DMA,
    ],
)


def _dot_t(a, b):
    return lax.dot_general(a, b, (((1,), (1,)), ((), ())),
                           preferred_element_type=jnp.float32)


def _embed_body(x_ref, w_ref, b_ref, o_ref):
    o_ref[...] = _dot_t(x_ref[...], w_ref[...]) + b_ref[...]


_embed = pl.pallas_call(
    _embed_body,
    out_shape=jax.ShapeDtypeStruct((N, H), jnp.float32),
)


def _conv_body(parts_ref, h_ref, wr_ref, br_ref, wroot_ref, g_ref, b_ref,
               o_ref):
    agg = parts_ref[0, :N, :] + parts_ref[1, :N, :]
    z = _dot_t(agg, wr_ref[...]) + _dot_t(h_ref[...], wroot_ref[...])
    z = z + br_ref[...]
    m = jnp.mean(z, axis=0, keepdims=True)
    v = jnp.mean((z - m) ** 2, axis=0, keepdims=True)
    o_ref[...] = jnp.maximum(
        (z - m) * lax.rsqrt(v + 1e-5) * g_ref[...] + b_ref[...], 0.0)


_conv = pl.pallas_call(
    _conv_body,
    out_shape=jax.ShapeDtypeStruct((N, H), jnp.float32),
)


def _final_body(parts_ref, h_ref, wr_ref, br_ref, wroot_ref, g_ref, b_ref,
                batch_ref, p1w_ref, p1b_ref, p2w_ref, p2b_ref, p3w_ref,
                p3b_ref, o_ref):
    agg = parts_ref[0, :N, :] + parts_ref[1, :N, :]
    z = _dot_t(agg, wr_ref[...]) + _dot_t(h_ref[...], wroot_ref[...])
    z = z + br_ref[...]
    m = jnp.mean(z, axis=0, keepdims=True)
    v = jnp.mean((z - m) ** 2, axis=0, keepdims=True)
    h2 = jnp.maximum(
        (z - m) * lax.rsqrt(v + 1e-5) * g_ref[...] + b_ref[...], 0.0)

    gids = lax.broadcasted_iota(jnp.int32, (N, G), 1)
    oh = (batch_ref[...] == gids).astype(jnp.float32)
    sums = lax.dot_general(oh, h2, (((0,), (0,)), ((), ())),
                           preferred_element_type=jnp.float32)
    cnt = jnp.sum(oh, axis=0)[:, None]
    gx = sums / jnp.maximum(cnt, 1.0)

    o = jnp.maximum(_dot_t(gx, p1w_ref[...]) + p1b_ref[...], 0.0)
    o = jnp.maximum(_dot_t(o, p2w_ref[...]) + p2b_ref[...], 0.0)
    o_ref[...] = (jnp.sum(o * p3w_ref[...], axis=1, keepdims=True)
                  + p3b_ref[...])


_final = pl.pallas_call(
    _final_body,
    out_shape=jax.ShapeDtypeStruct((G, 1), jnp.float32),
)


def kernel(x, edge_index, edge_attr, batch, emb_W, emb_b, c1_Wr, c1_br,
           c1_Wroot, bn1_g, bn1_b, c2_Wr, c2_br, c2_Wroot, bn2_g, bn2_b,
           p1_W, p1_b, p2_W, p2_b, p3_W, p3_b):
    pad = ((0, 0), (0, EPT_PAD - EPT))
    src3 = jnp.pad(edge_index[0].reshape(NW, EPT), pad).reshape(NW, NCH, CH)
    dst3 = jnp.pad(edge_index[1].reshape(NW, EPT), pad).reshape(NW, NCH, CH)
    ew3 = jnp.pad(edge_attr.reshape(NW, EPT), pad).reshape(NW, NCH, CH)

    batch2 = batch.reshape(N, 1)
    emb_b2 = emb_b.reshape(1, H)
    c1_br2 = c1_br.reshape(1, H)
    c2_br2 = c2_br.reshape(1, H)
    bn1_g2 = bn1_g.reshape(1, H)
    bn1_b2 = bn1_b.reshape(1, H)
    bn2_g2 = bn2_g.reshape(1, H)
    bn2_b2 = bn2_b.reshape(1, H)
    p1_b2 = p1_b.reshape(1, 2 * H)
    p2_b2 = p2_b.reshape(1, H)
    p3_b2 = p3_b.reshape(1, 1)

    h0 = _embed(x, emb_W, emb_b2)
    parts1 = _sc_agg(h0, src3, dst3, ew3)
    h1 = _conv(parts1, h0, c1_Wr, c1_br2, c1_Wroot, bn1_g2, bn1_b2)
    parts2 = _sc_agg(h1, src3, dst3, ew3)
    out = _final(parts2, h1, c2_Wr, c2_br2, c2_Wroot, bn2_g2, bn2_b2,
                 batch2, p1_W, p1_b2, p2_W, p2_b2, p3_W, p3_b2)
    return out


# speedup vs baseline: 3.7486x; 1.0500x over previous
import functools

import jax
import jax.numpy as jnp
from jax import lax
from jax.experimental import pallas as pl
from jax.experimental.pallas import tpu as pltpu
from jax.experimental.pallas import tpu_sc as plsc

N = 10000
E = 160000
IN = 256
H = 128
G = 64

NC = 2
NS = 16
NW = NC * NS
EPT = E // NW
CH = 128
NCH = (EPT + CH - 1) // CH
NCH += NCH % 2
EPT_PAD = NCH * CH
NP = 10240
STRIPE = NP // NS
LANES = 16


def _sc_agg_body(h_hbm, src_hbm, dst_hbm, ew_hbm, out_hbm,
                 src_v, dst_v, ew_v, gb0, gb1, agg_sh,
                 sem_g0, sem_g1, sem_s0, sem_s1):
    c = lax.axis_index("c")
    s = lax.axis_index("s")
    w = c * NS + s

    def zero_row(i, carry):
        for f in range(H // LANES):
            gb0[i, pl.ds(f * LANES, LANES)] = jnp.zeros((LANES,), jnp.float32)
        return carry
    lax.fori_loop(0, CH, zero_row, 0)
    for k in range(STRIPE // CH):
        pltpu.sync_copy(gb0,
                        agg_sh.at[pl.ds(s * STRIPE + k * CH, CH), :])

    pltpu.async_copy(src_hbm.at[w], src_v, sem_g0)
    pltpu.async_copy(dst_hbm.at[w], dst_v, sem_g1)
    pltpu.async_copy(ew_hbm.at[w], ew_v, sem_s0)
    pltpu.make_async_copy(src_hbm.at[w], src_v, sem_g0).wait()
    pltpu.make_async_copy(dst_hbm.at[w], dst_v, sem_g1).wait()
    pltpu.make_async_copy(ew_hbm.at[w], ew_v, sem_s0).wait()

    plsc.subcore_barrier()

    def scale(j, buf):
        def egroup(g, carry2):
            ew16 = ew_v[j, pl.ds(g * LANES, LANES)]
            for l in range(LANES):
                wgt = jnp.full((LANES,), ew16[l], jnp.float32)
                e = g * LANES + l
                for f in range(H // LANES):
                    sl = pl.ds(f * LANES, LANES)
                    buf[e, sl] = buf[e, sl] * wgt
            return carry2
        lax.fori_loop(0, CH // LANES, egroup, 0)

    def start_gather(j, gbuf, sem):
        pltpu.async_copy(h_hbm.at[src_v.at[j]], gbuf, sem)

    def wait_gather(j, gbuf, sem):
        pltpu.make_async_copy(h_hbm.at[src_v.at[j]], gbuf, sem).wait()

    def start_scatter(j, sbuf, sem):
        pltpu.async_copy(sbuf, agg_sh.at[dst_v.at[j]], sem, add=True)

    def wait_scatter(j, sbuf, sem):
        pltpu.make_async_copy(sbuf, agg_sh.at[dst_v.at[j]], sem).wait()

    start_gather(0, gb0, sem_g0)
    start_gather(1, gb1, sem_g1)

    def pair(t, carry):
        j0 = 2 * t
        j1 = j0 + 1

        wait_gather(j0, gb0, sem_g0)
        scale(j0, gb0)
        start_scatter(j0, gb0, sem_s0)

        wait_gather(j1, gb1, sem_g1)
        scale(j1, gb1)
        start_scatter(j1, gb1, sem_s1)

        @pl.when(t < NCH // 2 - 1)
        def _():
            wait_scatter(j0, gb0, sem_s0)
            start_gather(j0 + 2, gb0, sem_g0)
            wait_scatter(j1, gb1, sem_s1)
            start_gather(j1 + 2, gb1, sem_g1)
        return carry
    lax.fori_loop(0, NCH // 2, pair, 0)

    wait_scatter(NCH - 2, gb0, sem_s0)
    wait_scatter(NCH - 1, gb1, sem_s1)

    plsc.subcore_barrier()

    pltpu.sync_copy(agg_sh.at[pl.ds(s * STRIPE, STRIPE), :],
                    out_hbm.at[c, pl.ds(s * STRIPE, STRIPE), :])


_sc_agg = pl.kernel(
    _sc_agg_body,
    out_type=jax.ShapeDtypeStruct((NC, NP, H), jnp.float32),
    mesh=plsc.VectorSubcoreMesh(core_axis_name="c", subcore_axis_name="s"),
    scratch_types=[
        pltpu.VMEM((NCH, CH), jnp.int32),
        pltpu.VMEM((NCH, CH), jnp.int32),
        pltpu.VMEM((NCH, CH), jnp.float32),
        pltpu.VMEM((CH, H), jnp.float32),
        pltpu.VMEM((CH, H), jnp.float32),
        pltpu.VMEM_SHARED((NP, H), jnp.float32),
        pltpu.SemaphoreType.DMA,
        pltpu.SemaphoreType.DMA,
        pltpu.SemaphoreType.DMA,
        pltpu.SemaphoreType.DMA,
    ],
)


def _dot_t(a, b):
    return lax.dot_general(a, b, (((1,), (1,)), ((), ())),
                           preferred_element_type=jnp.float32)


def _embed_body(x_ref, w_ref, b_ref, o_ref):
    o_ref[...] = _dot_t(x_ref[...], w_ref[...]) + b_ref[...]


_embed = pl.pallas_call(
    _embed_body,
    out_shape=jax.ShapeDtypeStruct((N, H), jnp.float32),
)


def _conv_body(parts_ref, h_ref, wr_ref, br_ref, wroot_ref, g_ref, b_ref,
               o_ref):
    agg = parts_ref[0, :N, :] + parts_ref[1, :N, :]
    z = _dot_t(agg, wr_ref[...]) + _dot_t(h_ref[...], wroot_ref[...])
    z = z + br_ref[...]
    m = jnp.mean(z, axis=0, keepdims=True)
    v = jnp.mean((z - m) ** 2, axis=0, keepdims=True)
    o_ref[...] = jnp.maximum(
        (z - m) * lax.rsqrt(v + 1e-5) * g_ref[...] + b_ref[...], 0.0)


_conv = pl.pallas_call(
    _conv_body,
    out_shape=jax.ShapeDtypeStruct((N, H), jnp.float32),
)


def _final_body(parts_ref, h_ref, wr_ref, br_ref, wroot_ref, g_ref, b_ref,
                batch_ref, p1w_ref, p1b_ref, p2w_ref, p2b_ref, p3w_ref,
                p3b_ref, o_ref):
    agg = parts_ref[0, :N, :] + parts_ref[1, :N, :]
    z = _dot_t(agg, wr_ref[...]) + _dot_t(h_ref[...], wroot_ref[...])
    z = z + br_ref[...]
    m = jnp.mean(z, axis=0, keepdims=True)
    v = jnp.mean((z - m) ** 2, axis=0, keepdims=True)
    h2 = jnp.maximum(
        (z - m) * lax.rsqrt(v + 1e-5) * g_ref[...] + b_ref[...], 0.0)

    gids = lax.broadcasted_iota(jnp.int32, (N, G), 1)
    oh = (batch_ref[...] == gids).astype(jnp.float32)
    sums = lax.dot_general(oh, h2, (((0,), (0,)), ((), ())),
                           preferred_element_type=jnp.float32)
    cnt = jnp.sum(oh, axis=0)[:, None]
    gx = sums / jnp.maximum(cnt, 1.0)

    o = jnp.maximum(_dot_t(gx, p1w_ref[...]) + p1b_ref[...], 0.0)
    o = jnp.maximum(_dot_t(o, p2w_ref[...]) + p2b_ref[...], 0.0)
    o_ref[...] = (jnp.sum(o * p3w_ref[...], axis=1, keepdims=True)
                  + p3b_ref[...])


_final = pl.pallas_call(
    _final_body,
    out_shape=jax.ShapeDtypeStruct((G, 1), jnp.float32),
)


def kernel(x, edge_index, edge_attr, batch, emb_W, emb_b, c1_Wr, c1_br,
           c1_Wroot, bn1_g, bn1_b, c2_Wr, c2_br, c2_Wroot, bn2_g, bn2_b,
           p1_W, p1_b, p2_W, p2_b, p3_W, p3_b):
    pad = ((0, 0), (0, EPT_PAD - EPT))
    src3 = jnp.pad(edge_index[0].reshape(NW, EPT), pad).reshape(NW, NCH, CH)
    dst3 = jnp.pad(edge_index[1].reshape(NW, EPT), pad).reshape(NW, NCH, CH)
    ew3 = jnp.pad(edge_attr.reshape(NW, EPT), pad).reshape(NW, NCH, CH)

    batch2 = batch.reshape(N, 1)
    emb_b2 = emb_b.reshape(1, H)
    c1_br2 = c1_br.reshape(1, H)
    c2_br2 = c2_br.reshape(1, H)
    bn1_g2 = bn1_g.reshape(1, H)
    bn1_b2 = bn1_b.reshape(1, H)
    bn2_g2 = bn2_g.reshape(1, H)
    bn2_b2 = bn2_b.reshape(1, H)
    p1_b2 = p1_b.reshape(1, 2 * H)
    p2_b2 = p2_b.reshape(1, H)
    p3_b2 = p3_b.reshape(1, 1)

    h0 = _embed(x, emb_W, emb_b2)
    parts1 = _sc_agg(h0, src3, dst3, ew3)
    h1 = _conv(parts1, h0, c1_Wr, c1_br2, c1_Wroot, bn1_g2, bn1_b2)
    parts2 = _sc_agg(h1, src3, dst3, ew3)
    out = _final(parts2, h1, c2_Wr, c2_br2, c2_Wroot, bn2_g2, bn2_b2,
                 batch2, p1_W, p1_b2, p2_W, p2_b2, p3_W, p3_b2)
    return out
